# baseline (speedup 1.0000x reference)
import jax
import jax.numpy as jnp
from jax.experimental import pallas as pl
from jax.experimental.pallas import tpu as pltpu

_BM = 400
_SUB = 2
_BR = _BM // _SUB


def _gcn_block(seq_ref, adj_ref, w_ref, a_ref, out_ref):
    r = pl.program_id(0) % _SUB
    h = jnp.dot(adj_ref[pl.ds(r * _BR, _BR), :].astype(jnp.bfloat16),
                seq_ref[...].astype(jnp.bfloat16),
                preferred_element_type=jnp.float32)
    y = jax.lax.dot_general(h, w_ref[...], (((1,), (1,)), ((), ())),
                            preferred_element_type=jnp.float32)
    slope = a_ref[0, 0]
    out_ref[...] = jnp.where(y >= 0, y, slope * y)


def kernel(seq, adj, W, a):
    N, d_in = seq.shape
    d_out = W.shape[0]
    return pl.pallas_call(
        _gcn_block,
        grid=(N // _BR,),
        in_specs=[
            pl.BlockSpec((N, d_in), lambda i: (0, 0)),
            pl.BlockSpec((_BM, N), lambda i: (i // _SUB, 0)),
            pl.BlockSpec((d_out, d_in), lambda i: (0, 0)),
            pl.BlockSpec(memory_space=pltpu.SMEM),
        ],
        out_specs=pl.BlockSpec((_BR, d_out), lambda i: (i, 0)),
        out_shape=jax.ShapeDtypeStruct((N, d_out), jnp.float32),
    )(seq, adj, W, a.reshape(1, 1))


# speedup vs baseline: 1.3666x; 1.3666x over previous
import jax
import jax.numpy as jnp
from jax.experimental import pallas as pl
from jax.experimental.pallas import tpu as pltpu

_N = 10000
_DIN = 128
_DOUT = 128
_BM = 416
_BLOCKS = [_BM] * 23 + [216, 104, 56, 32, 16, 8]
_OFFS = [0]
for _b in _BLOCKS:
    _OFFS.append(_OFFS[-1] + _b)
assert _OFFS[-1] == _N


def _gcn_body(seq_ref, adj_ref, w_ref, a_ref, out_ref,
              buf0, buf1, seqb, outb0, outb1,
              sem0, sem1, osem0, osem1):
    bufs = (buf0, buf1)
    outbs = (outb0, outb1)
    sems = (sem0, sem1)
    osems = (osem0, osem1)
    nb = len(_BLOCKS)

    def in_copy(r):
        return pltpu.make_async_copy(
            adj_ref.at[pl.ds(_OFFS[r], _BLOCKS[r])],
            bufs[r % 2].at[pl.ds(0, _BLOCKS[r])],
            sems[r % 2])

    def out_copy(r):
        return pltpu.make_async_copy(
            outbs[r % 2].at[pl.ds(0, _BLOCKS[r])],
            out_ref.at[pl.ds(_OFFS[r], _BLOCKS[r])],
            osems[r % 2])

    in_copy(0).start()
    in_copy(1).start()
    seqb[...] = seq_ref[...].astype(jnp.bfloat16)
    wb = w_ref[...]
    slope = a_ref[0, 0]

    for r in range(nb):
        in_copy(r).wait()
        h = jnp.dot(bufs[r % 2][: _BLOCKS[r], :].astype(jnp.bfloat16),
                    seqb[...], preferred_element_type=jnp.float32)
        if r + 2 < nb:
            in_copy(r + 2).start()
        y = jax.lax.dot_general(h, wb, (((1,), (1,)), ((), ())),
                                preferred_element_type=jnp.float32)
        y = jnp.where(y >= 0, y, slope * y)
        if r >= 2:
            out_copy(r - 2).wait()
        outbs[r % 2][: _BLOCKS[r], :] = y
        out_copy(r).start()

    out_copy(nb - 2).wait()
    out_copy(nb - 1).wait()


def kernel(seq, adj, W, a):
    N, d_in = seq.shape
    d_out = W.shape[0]
    return pl.pallas_call(
        _gcn_body,
        in_specs=[
            pl.BlockSpec((N, d_in), lambda: (0, 0)),
            pl.BlockSpec(memory_space=pl.ANY),
            pl.BlockSpec((d_out, d_in), lambda: (0, 0)),
            pl.BlockSpec(memory_space=pltpu.SMEM),
        ],
        out_specs=pl.BlockSpec(memory_space=pl.ANY),
        out_shape=jax.ShapeDtypeStruct((N, d_out), jnp.float32),
        scratch_shapes=[
            pltpu.VMEM((_BM, _N), jnp.float32),
            pltpu.VMEM((_BM, _N), jnp.float32),
            pltpu.VMEM((_N, _DIN), jnp.bfloat16),
            pltpu.VMEM((_BM, _DOUT), jnp.float32),
            pltpu.VMEM((_BM, _DOUT), jnp.float32),
            pltpu.SemaphoreType.DMA,
            pltpu.SemaphoreType.DMA,
            pltpu.SemaphoreType.DMA,
            pltpu.SemaphoreType.DMA,
        ],
    )(seq, adj, W, a.reshape(1, 1))
